# baseline (speedup 1.0000x reference)
import functools

import jax
import jax.numpy as jnp
from jax import lax
from jax.experimental import pallas as pl
from jax.experimental.pallas import tpu as pltpu
from jax.experimental.pallas import tpu_sc as plsc

NC = 2
NS = 16
NW = NC * NS
LANES = 16
KD = 128
KS = 128


def _zero_rows(ref, n_rows, n_cols):
    zeros16 = jnp.zeros((LANES,), jnp.float32)

    @pl.loop(0, n_rows)
    def _(i):
        for j in range(n_cols // LANES):
            ref[i, pl.ds(j * LANES, LANES)] = zeros16


def _fill_1d(ref, n, value):
    vec = jnp.full((LANES,), value, jnp.float32)
    for j in range(n // LANES):
        ref[pl.ds(j * LANES, LANES)] = vec


def _make_deg_kernel(n_chunks, n_pad):
    ch = n_chunks // NW
    rows_per_tile = n_pad // NS
    mesh = plsc.VectorSubcoreMesh(core_axis_name="c", subcore_axis_name="s")

    @functools.partial(
        pl.kernel,
        out_type=jax.ShapeDtypeStruct((NC, n_pad), jnp.float32),
        mesh=mesh,
        scratch_types=[
            pltpu.VMEM((ch, KD), jnp.int32),
            pltpu.VMEM((KD,), jnp.float32),
            pltpu.VMEM((KD,), jnp.float32),
            pltpu.VMEM_SHARED((n_pad,), jnp.float32),
            pltpu.SemaphoreType.DMA,
        ],
    )
    def deg_kernel(dst_hbm, out_hbm, dst2d, ones_v, zero_v, acc_sh, sem):
        c = lax.axis_index("c")
        s = lax.axis_index("s")
        wid = s * NC + c
        _fill_1d(ones_v, KD, 1.0)
        _fill_1d(zero_v, KD, 0.0)
        for j in range(rows_per_tile // KD):
            pltpu.sync_copy(zero_v, acc_sh.at[pl.ds(s * rows_per_tile + j * KD, KD)])
        pltpu.sync_copy(dst_hbm.at[pl.ds(wid * ch, ch)], dst2d)
        plsc.subcore_barrier()

        @pl.loop(0, ch)
        def _(i):
            pltpu.async_copy(ones_v, acc_sh.at[dst2d.at[i]], sem, add=True)

        pltpu.make_async_copy(dst_hbm.at[pl.ds(0, ch)], dst2d, sem).wait()

        plsc.subcore_barrier()
        pltpu.sync_copy(acc_sh.at[pl.ds(s * rows_per_tile, rows_per_tile)],
                        out_hbm.at[c, pl.ds(s * rows_per_tile, rows_per_tile)])

    return deg_kernel


def _make_scatter_kernel(n_chunks, n_pad, d):
    ch = n_chunks // NW
    assert ch % 2 == 0 and ch >= 4
    rows_per_tile = n_pad // NS
    mesh = plsc.VectorSubcoreMesh(core_axis_name="c", subcore_axis_name="s")

    @functools.partial(
        pl.kernel,
        out_type=jax.ShapeDtypeStruct((NC, n_pad, d), jnp.float32),
        mesh=mesh,
        scratch_types=[
            pltpu.VMEM((2, KS), jnp.int32),
            pltpu.VMEM((2, KS), jnp.int32),
            pltpu.VMEM((KS, d), jnp.float32),
            pltpu.VMEM((KS, d), jnp.float32),
            pltpu.VMEM_SHARED((n_pad, d), jnp.float32),
            pltpu.SemaphoreType.DMA,
            pltpu.SemaphoreType.DMA,
            pltpu.SemaphoreType.DMA,
            pltpu.SemaphoreType.DMA,
        ],
    )
    def scatter_kernel(hs_hbm, src_hbm, dst_hbm, out_hbm,
                       idx_a, idx_b, rows_a, rows_b,
                       acc_sh, sem_ia, sem_ib, sem_ga, sem_gb):
        c = lax.axis_index("c")
        s = lax.axis_index("s")
        wid = s * NC + c
        base = wid * ch

        _zero_rows(rows_a, KS, d)
        for j in range(rows_per_tile // KS):
            pltpu.sync_copy(rows_a, acc_sh.at[pl.ds(s * rows_per_tile + j * KS, KS)])
        plsc.subcore_barrier()

        def start_idx(idx, sem, i):
            off = (base + i) * KS
            pltpu.async_copy(src_hbm.at[pl.ds(off, KS)], idx.at[0], sem)
            pltpu.async_copy(dst_hbm.at[pl.ds(off, KS)], idx.at[1], sem)

        def wait_idx(idx, sem):
            pltpu.make_async_copy(src_hbm.at[pl.ds(0, KS)], idx.at[0], sem).wait()
            pltpu.make_async_copy(src_hbm.at[pl.ds(0, KS)], idx.at[1], sem).wait()

        def start_gather(idx, buf, sem):
            pltpu.async_copy(hs_hbm.at[idx.at[0]], buf, sem)

        def wait_gather(buf, sem):
            pltpu.make_async_copy(hs_hbm.at[pl.ds(0, KS)], buf, sem).wait()

        def scatter(idx, buf):
            pltpu.sync_copy(buf, acc_sh.at[idx.at[1]], add=True)

        start_idx(idx_a, sem_ia, 0)
        start_idx(idx_b, sem_ib, 1)
        wait_idx(idx_a, sem_ia)
        start_gather(idx_a, rows_a, sem_ga)
        wait_idx(idx_b, sem_ib)
        start_gather(idx_b, rows_b, sem_gb)

        @pl.loop(0, ch // 2 - 1)
        def _(j):
            i0 = 2 * j
            wait_gather(rows_a, sem_ga)
            scatter(idx_a, rows_a)
            start_idx(idx_a, sem_ia, i0 + 2)
            wait_gather(rows_b, sem_gb)
            wait_idx(idx_a, sem_ia)
            start_gather(idx_a, rows_a, sem_ga)
            scatter(idx_b, rows_b)
            start_idx(idx_b, sem_ib, i0 + 3)
            wait_idx(idx_b, sem_ib)
            start_gather(idx_b, rows_b, sem_gb)

        wait_gather(rows_a, sem_ga)
        scatter(idx_a, rows_a)
        wait_gather(rows_b, sem_gb)
        scatter(idx_b, rows_b)

        plsc.subcore_barrier()
        pltpu.sync_copy(acc_sh.at[pl.ds(s * rows_per_tile, rows_per_tile)],
                        out_hbm.at[c, pl.ds(s * rows_per_tile, rows_per_tile)])

    return scatter_kernel


def _dinv_block(degp_ref):
    deg = degp_ref[0, :] + degp_ref[1, :] + 1.0
    return lax.rsqrt(deg)


def _tc_first(x, w1, degp, r, n_pad):
    n, d_in = x.shape
    d_h = w1.shape[1]

    def body(x_ref, w_ref, degp_ref, o_ref):
        h = jnp.dot(x_ref[...], w_ref[...], preferred_element_type=jnp.float32)
        o_ref[...] = h * _dinv_block(degp_ref)[:, None]

    return pl.pallas_call(
        body,
        grid=(n_pad // r,),
        in_specs=[
            pl.BlockSpec((r, d_in), lambda i: (i, 0)),
            pl.BlockSpec((d_in, d_h), lambda i: (0, 0)),
            pl.BlockSpec((NC, r), lambda i: (0, i)),
        ],
        out_specs=pl.BlockSpec((r, d_h), lambda i: (i, 0)),
        out_shape=jax.ShapeDtypeStruct((n_pad, d_h), jnp.float32),
    )(x, w1, degp)


def _tc_mid(s1, hs1, degp, b1, w2, r):
    n_pad, d_h = hs1.shape
    d_o = w2.shape[1]

    def body(s1_ref, hs_ref, degp_ref, b_ref, w_ref, o_ref):
        dinv = _dinv_block(degp_ref)[:, None]
        conv = dinv * (s1_ref[0] + s1_ref[1] + hs_ref[...]) + b_ref[...]
        h = jnp.maximum(conv, 0.0)
        o_ref[...] = jnp.dot(h, w_ref[...], preferred_element_type=jnp.float32) * dinv

    return pl.pallas_call(
        body,
        grid=(n_pad // r,),
        in_specs=[
            pl.BlockSpec((NC, r, d_h), lambda i: (0, i, 0)),
            pl.BlockSpec((r, d_h), lambda i: (i, 0)),
            pl.BlockSpec((NC, r), lambda i: (0, i)),
            pl.BlockSpec((1, d_h), lambda i: (0, 0)),
            pl.BlockSpec((d_h, d_o), lambda i: (0, 0)),
        ],
        out_specs=pl.BlockSpec((r, d_o), lambda i: (i, 0)),
        out_shape=jax.ShapeDtypeStruct((n_pad, d_o), jnp.float32),
    )(s1, hs1, degp, b1, w2)


def _tc_final(s2, hs2, degp, b2, r, n):
    _, d_o = hs2.shape

    def body(s2_ref, hs_ref, degp_ref, b_ref, o_ref):
        dinv = _dinv_block(degp_ref)[:, None]
        conv = dinv * (s2_ref[0] + s2_ref[1] + hs_ref[...]) + b_ref[...]
        o_ref[...] = jnp.maximum(conv, 0.0)

    return pl.pallas_call(
        body,
        grid=(pl.cdiv(n, r),),
        in_specs=[
            pl.BlockSpec((NC, r, d_o), lambda i: (0, i, 0)),
            pl.BlockSpec((r, d_o), lambda i: (i, 0)),
            pl.BlockSpec((NC, r), lambda i: (0, i)),
            pl.BlockSpec((1, d_o), lambda i: (0, 0)),
        ],
        out_specs=pl.BlockSpec((r, d_o), lambda i: (i, 0)),
        out_shape=jax.ShapeDtypeStruct((n, d_o), jnp.float32),
    )(s2, hs2, degp, b2)


def kernel(x, edge_index, W1, b1, W2, b2):
    n, d_in = x.shape
    e_total = edge_index.shape[1]
    d_h = W1.shape[1]
    d_o = W2.shape[1]
    n_pad = ((n + NW * LANES - 1) // (NW * LANES)) * (NW * LANES)

    unit = NW * KD * 8
    e_pad = ((e_total + unit - 1) // unit) * unit
    sentinel = jnp.full((e_pad - e_total,), n_pad - 1, jnp.int32)
    src_flat = jnp.concatenate([edge_index[0].astype(jnp.int32), sentinel])
    dst_flat = jnp.concatenate([edge_index[1].astype(jnp.int32), sentinel])
    b1r = b1.reshape(1, d_h).astype(jnp.float32)
    b2r = b2.reshape(1, d_o).astype(jnp.float32)

    r = 1024

    degp = _make_deg_kernel(e_pad // KD, n_pad)(dst_flat.reshape(-1, KD))
    scat = _make_scatter_kernel(e_pad // KS, n_pad, d_h)
    src = src_flat
    dst = dst_flat

    hs1 = _tc_first(x, W1.astype(jnp.float32), degp, r, n_pad)
    s1 = scat(hs1, src, dst)
    hs2 = _tc_mid(s1, hs1, degp, b1r, W2.astype(jnp.float32), r)
    s2 = scat(hs2, src, dst)
    out = _tc_final(s2, hs2, degp, b2r, r, n)
    return out


# speedup vs baseline: 1.0004x; 1.0004x over previous
import functools

import jax
import jax.numpy as jnp
from jax import lax
from jax.experimental import pallas as pl
from jax.experimental.pallas import tpu as pltpu
from jax.experimental.pallas import tpu_sc as plsc

NC = 2
NS = 16
NW = NC * NS
LANES = 16
KD = 128
KS = 128


def _zero_rows(ref, n_rows, n_cols):
    zeros16 = jnp.zeros((LANES,), jnp.float32)

    @pl.loop(0, n_rows)
    def _(i):
        for j in range(n_cols // LANES):
            ref[i, pl.ds(j * LANES, LANES)] = zeros16


def _fill_1d(ref, n, value):
    vec = jnp.full((LANES,), value, jnp.float32)
    for j in range(n // LANES):
        ref[pl.ds(j * LANES, LANES)] = vec


def _make_deg_kernel(n_chunks, n_pad):
    ch = n_chunks // NW
    rows_per_tile = n_pad // NS
    mesh = plsc.VectorSubcoreMesh(core_axis_name="c", subcore_axis_name="s")

    @functools.partial(
        pl.kernel,
        out_type=jax.ShapeDtypeStruct((NC, n_pad), jnp.float32),
        mesh=mesh,
        scratch_types=[
            pltpu.VMEM((ch, KD), jnp.int32),
            pltpu.VMEM((KD,), jnp.float32),
            pltpu.VMEM((KD,), jnp.float32),
            pltpu.VMEM_SHARED((n_pad,), jnp.float32),
            pltpu.SemaphoreType.DMA,
        ],
    )
    def deg_kernel(dst_hbm, out_hbm, dst2d, ones_v, zero_v, acc_sh, sem):
        c = lax.axis_index("c")
        s = lax.axis_index("s")
        wid = s * NC + c
        _fill_1d(ones_v, KD, 1.0)
        _fill_1d(zero_v, KD, 0.0)
        for j in range(rows_per_tile // KD):
            pltpu.sync_copy(zero_v, acc_sh.at[pl.ds(s * rows_per_tile + j * KD, KD)])
        pltpu.sync_copy(dst_hbm.at[pl.ds(wid * ch, ch)], dst2d)
        plsc.subcore_barrier()

        @pl.loop(0, ch)
        def _(i):
            pltpu.async_copy(ones_v, acc_sh.at[dst2d.at[i]], sem, add=True)

        pltpu.make_async_copy(dst_hbm.at[pl.ds(0, ch)], dst2d, sem).wait()

        plsc.subcore_barrier()
        pltpu.sync_copy(acc_sh.at[pl.ds(s * rows_per_tile, rows_per_tile)],
                        out_hbm.at[c, pl.ds(s * rows_per_tile, rows_per_tile)])

    return deg_kernel


def _make_scatter_kernel(n_chunks, n_pad, d):
    ch = n_chunks // NW
    assert ch % 2 == 0 and ch >= 4
    rows_per_tile = n_pad // NS
    mesh = plsc.VectorSubcoreMesh(core_axis_name="c", subcore_axis_name="s")

    @functools.partial(
        pl.kernel,
        out_type=jax.ShapeDtypeStruct((NC, n_pad, d), jnp.float32),
        mesh=mesh,
        scratch_types=[
            pltpu.VMEM((KS,), jnp.int32),
            pltpu.VMEM((KS,), jnp.int32),
            pltpu.VMEM((KS,), jnp.int32),
            pltpu.VMEM((KS,), jnp.int32),
            pltpu.VMEM((KS, d), jnp.float32),
            pltpu.VMEM((KS, d), jnp.float32),
            pltpu.VMEM_SHARED((n_pad, d), jnp.float32),
            pltpu.SemaphoreType.DMA,
            pltpu.SemaphoreType.DMA,
            pltpu.SemaphoreType.DMA,
            pltpu.SemaphoreType.DMA,
        ],
    )
    def scatter_kernel(hs_hbm, src_hbm, dst_hbm, out_hbm,
                       src_a, dst_a, src_b, dst_b, rows_a, rows_b,
                       acc_sh, sem_ia, sem_ib, sem_ga, sem_gb):
        c = lax.axis_index("c")
        s = lax.axis_index("s")
        wid = s * NC + c
        base = wid * ch

        _zero_rows(rows_a, KS, d)
        for j in range(rows_per_tile // KS):
            pltpu.sync_copy(rows_a, acc_sh.at[pl.ds(s * rows_per_tile + j * KS, KS)])
        plsc.subcore_barrier()

        def start_idx(idx_s, idx_d, sem, i):
            off = (base + i) * KS
            pltpu.async_copy(src_hbm.at[pl.ds(off, KS)], idx_s, sem)
            pltpu.async_copy(dst_hbm.at[pl.ds(off, KS)], idx_d, sem)

        def wait_idx(idx_s, idx_d, sem):
            pltpu.make_async_copy(src_hbm.at[pl.ds(0, KS)], idx_s, sem).wait()
            pltpu.make_async_copy(src_hbm.at[pl.ds(0, KS)], idx_d, sem).wait()

        def start_gather(idx_s, buf, sem):
            pltpu.async_copy(hs_hbm.at[idx_s], buf, sem)

        def wait_gather(buf, sem):
            pltpu.make_async_copy(hs_hbm.at[pl.ds(0, KS)], buf, sem).wait()

        def scatter(idx_d, buf):
            pltpu.sync_copy(buf, acc_sh.at[idx_d], add=True)

        start_idx(src_a, dst_a, sem_ia, 0)
        start_idx(src_b, dst_b, sem_ib, 1)
        wait_idx(src_a, dst_a, sem_ia)
        start_gather(src_a, rows_a, sem_ga)
        wait_idx(src_b, dst_b, sem_ib)
        start_gather(src_b, rows_b, sem_gb)

        @pl.loop(0, ch // 2 - 1)
        def _(j):
            i0 = 2 * j
            wait_gather(rows_a, sem_ga)
            scatter(dst_a, rows_a)
            start_idx(src_a, dst_a, sem_ia, i0 + 2)
            wait_gather(rows_b, sem_gb)
            wait_idx(src_a, dst_a, sem_ia)
            start_gather(src_a, rows_a, sem_ga)
            scatter(dst_b, rows_b)
            start_idx(src_b, dst_b, sem_ib, i0 + 3)
            wait_idx(src_b, dst_b, sem_ib)
            start_gather(src_b, rows_b, sem_gb)

        wait_gather(rows_a, sem_ga)
        scatter(dst_a, rows_a)
        wait_gather(rows_b, sem_gb)
        scatter(dst_b, rows_b)

        plsc.subcore_barrier()
        pltpu.sync_copy(acc_sh.at[pl.ds(s * rows_per_tile, rows_per_tile)],
                        out_hbm.at[c, pl.ds(s * rows_per_tile, rows_per_tile)])

    return scatter_kernel


def _dinv_block(degp_ref):
    deg = degp_ref[0, :] + degp_ref[1, :] + 1.0
    return lax.rsqrt(deg)


def _tc_first(x, w1, degp, r, n_pad):
    n, d_in = x.shape
    d_h = w1.shape[1]

    def body(x_ref, w_ref, degp_ref, o_ref):
        h = jnp.dot(x_ref[...], w_ref[...], preferred_element_type=jnp.float32)
        o_ref[...] = h * _dinv_block(degp_ref)[:, None]

    return pl.pallas_call(
        body,
        grid=(n_pad // r,),
        in_specs=[
            pl.BlockSpec((r, d_in), lambda i: (i, 0)),
            pl.BlockSpec((d_in, d_h), lambda i: (0, 0)),
            pl.BlockSpec((NC, r), lambda i: (0, i)),
        ],
        out_specs=pl.BlockSpec((r, d_h), lambda i: (i, 0)),
        out_shape=jax.ShapeDtypeStruct((n_pad, d_h), jnp.float32),
    )(x, w1, degp)


def _tc_mid(s1, hs1, degp, b1, w2, r):
    n_pad, d_h = hs1.shape
    d_o = w2.shape[1]

    def body(s1_ref, hs_ref, degp_ref, b_ref, w_ref, o_ref):
        dinv = _dinv_block(degp_ref)[:, None]
        conv = dinv * (s1_ref[0] + s1_ref[1] + hs_ref[...]) + b_ref[...]
        h = jnp.maximum(conv, 0.0)
        o_ref[...] = jnp.dot(h, w_ref[...], preferred_element_type=jnp.float32) * dinv

    return pl.pallas_call(
        body,
        grid=(n_pad // r,),
        in_specs=[
            pl.BlockSpec((NC, r, d_h), lambda i: (0, i, 0)),
            pl.BlockSpec((r, d_h), lambda i: (i, 0)),
            pl.BlockSpec((NC, r), lambda i: (0, i)),
            pl.BlockSpec((1, d_h), lambda i: (0, 0)),
            pl.BlockSpec((d_h, d_o), lambda i: (0, 0)),
        ],
        out_specs=pl.BlockSpec((r, d_o), lambda i: (i, 0)),
        out_shape=jax.ShapeDtypeStruct((n_pad, d_o), jnp.float32),
    )(s1, hs1, degp, b1, w2)


def _tc_final(s2, hs2, degp, b2, r, n):
    _, d_o = hs2.shape

    def body(s2_ref, hs_ref, degp_ref, b_ref, o_ref):
        dinv = _dinv_block(degp_ref)[:, None]
        conv = dinv * (s2_ref[0] + s2_ref[1] + hs_ref[...]) + b_ref[...]
        o_ref[...] = jnp.maximum(conv, 0.0)

    return pl.pallas_call(
        body,
        grid=(pl.cdiv(n, r),),
        in_specs=[
            pl.BlockSpec((NC, r, d_o), lambda i: (0, i, 0)),
            pl.BlockSpec((r, d_o), lambda i: (i, 0)),
            pl.BlockSpec((NC, r), lambda i: (0, i)),
            pl.BlockSpec((1, d_o), lambda i: (0, 0)),
        ],
        out_specs=pl.BlockSpec((r, d_o), lambda i: (i, 0)),
        out_shape=jax.ShapeDtypeStruct((n, d_o), jnp.float32),
    )(s2, hs2, degp, b2)


def kernel(x, edge_index, W1, b1, W2, b2):
    n, d_in = x.shape
    e_total = edge_index.shape[1]
    d_h = W1.shape[1]
    d_o = W2.shape[1]
    n_pad = ((n + NW * LANES - 1) // (NW * LANES)) * (NW * LANES)

    unit = NW * KD * 8
    e_pad = ((e_total + unit - 1) // unit) * unit
    sentinel = jnp.full((e_pad - e_total,), n_pad - 1, jnp.int32)
    src_flat = jnp.concatenate([edge_index[0].astype(jnp.int32), sentinel])
    dst_flat = jnp.concatenate([edge_index[1].astype(jnp.int32), sentinel])
    b1r = b1.reshape(1, d_h).astype(jnp.float32)
    b2r = b2.reshape(1, d_o).astype(jnp.float32)

    r = 1024

    degp = _make_deg_kernel(e_pad // KD, n_pad)(dst_flat.reshape(-1, KD))
    scat = _make_scatter_kernel(e_pad // KS, n_pad, d_h)
    src = src_flat
    dst = dst_flat

    hs1 = _tc_first(x, W1.astype(jnp.float32), degp, r, n_pad)
    s1 = scat(hs1, src, dst)
    hs2 = _tc_mid(s1, hs1, degp, b1r, W2.astype(jnp.float32), r)
    s2 = scat(hs2, src, dst)
    out = _tc_final(s2, hs2, degp, b2r, r, n)
    return out
